# baseline (speedup 1.0000x reference)
import jax
import jax.numpy as jnp
from jax.experimental import pallas as pl
from jax.experimental.pallas import tpu as pltpu

TOPK = 20
NEG = float("-inf")


def _body(hist_ref, out_ref, norm_ref):
    r = pl.program_id(1)
    br = out_ref.shape[0]
    n = out_ref.shape[1]
    w = hist_ref.shape[1]

    @pl.when(r == 0)
    def _normalize():
        h = hist_ref[...]
        mean = jnp.mean(h, axis=1, keepdims=True)
        c = h - mean
        denom = jnp.sqrt(jnp.mean(c * c, axis=1, keepdims=True))
        denom = jnp.maximum(denom, 1e-6)
        norm_ref[...] = c / denom

    rows = norm_ref[pl.ds(r * br, br), :]
    allr = norm_ref[...]
    sim = jax.lax.dot_general(
        rows, allr, (((1,), (1,)), ((), ())),
        preferred_element_type=jnp.float32) * (1.0 / w)
    sim = jnp.maximum(sim, 0.0)
    row_ids = r * br + jax.lax.broadcasted_iota(jnp.int32, (br, n), 0)
    col_ids = jax.lax.broadcasted_iota(jnp.int32, (br, n), 1)
    sim = jnp.where(row_ids == col_ids, 0.0, sim)

    g = n // 128
    depth = 4
    C = [sim[:, i * 128:(i + 1) * 128] for i in range(g)]
    for (i, j) in [(0, 1), (2, 3), (4, 5), (6, 7),
                   (0, 2), (1, 3), (4, 6), (5, 7),
                   (1, 2), (5, 6),
                   (0, 4), (1, 5), (2, 6), (3, 7),
                   (2, 4), (3, 5),
                   (1, 2), (3, 4)]:
        hi = jnp.maximum(C[i], C[j])
        lo = jnp.minimum(C[i], C[j])
        C[i], C[j] = hi, lo
    C = C[:depth]

    m = jnp.max(C[0], axis=1, keepdims=True)
    for _ in range(TOPK - 1):
        popped = C[0] >= m
        for i in range(depth - 1):
            C[i] = jnp.where(popped, C[i + 1], C[i])
        C[depth - 1] = jnp.where(popped, NEG, C[depth - 1])
        m = jnp.max(C[0], axis=1, keepdims=True)

    sparse = jnp.where(sim >= m, sim, 0.0)
    s1 = jnp.sum(sparse, axis=1, keepdims=True)
    r1 = 1.0 / jnp.maximum(s1, 1e-6)
    s2 = jnp.maximum(s1 * r1, 1e-6)
    out_ref[...] = sparse * (r1 / s2)


@jax.jit
def kernel(history):
    bsz, n, w = history.shape
    br = 512
    grid = (bsz, n // br)
    out = pl.pallas_call(
        _body,
        grid=grid,
        in_specs=[pl.BlockSpec((None, n, w), lambda b, r: (b, 0, 0))],
        out_specs=pl.BlockSpec((None, br, n), lambda b, r: (b, r, 0)),
        out_shape=jax.ShapeDtypeStruct((bsz, n, n), jnp.float32),
        scratch_shapes=[pltpu.VMEM((n, w), jnp.float32)],
    )(history)
    return out
